# baseline (speedup 1.0000x reference)
import functools

import jax
import jax.numpy as jnp
from jax import lax
from jax.experimental import pallas as pl
from jax.experimental.pallas import tpu as pltpu
from jax.experimental.pallas import tpu_sc as plsc

N_NODES = 10000
D = 128
DQ = D // 4
DH = D // 2
N_EDGES = 320000

NC = 2
NS = 16
CH = 128

E_TOT = N_EDGES + N_NODES
NCH = -(-E_TOT // (NS * CH))
NCH = -(-NCH // 4) * 4
E_PAD = NS * NCH * CH
N_PAD = -(-N_NODES // (NS * 8)) * NS * 8
ROWS_PT = N_PAD // NS

_sc_mesh = plsc.VectorSubcoreMesh(core_axis_name="c", subcore_axis_name="s")


def _deg_body(dsts_hbm, zeros_hbm, ones_hbm, deg_hbm, dstv, onesv, acc):
    c = lax.axis_index("c")
    s = lax.axis_index("s")
    pltpu.sync_copy(zeros_hbm, acc.at[pl.ds(s * ROWS_PT, ROWS_PT)])
    pltpu.sync_copy(dsts_hbm.at[s], dstv)
    pltpu.sync_copy(ones_hbm, onesv)
    plsc.subcore_barrier()

    base = c * (NCH // 2)

    def body(i, carry):
        pltpu.sync_copy(onesv, acc.at[dstv.at[base + i]], add=True)
        return carry

    lax.fori_loop(0, NCH // 2, body, 0, unroll=False)
    plsc.subcore_barrier()
    pltpu.sync_copy(acc.at[pl.ds(s * ROWS_PT, ROWS_PT)],
                    deg_hbm.at[c, pl.ds(s * ROWS_PT, ROWS_PT)])


def _make_deg(interpret=False):
    return pl.kernel(
        _deg_body,
        out_type=jax.ShapeDtypeStruct((NC, N_PAD, 16), jnp.float32),
        mesh=_sc_mesh,
        scratch_types=[
            pltpu.VMEM((NCH, CH), jnp.int32),
            pltpu.VMEM((CH, 16), jnp.float32),
            pltpu.VMEM_SHARED((N_PAD, 16), jnp.float32),
        ],
        compiler_params=pltpu.CompilerParams(use_tc_tiling_on_sc=False),
        interpret=interpret,
    )


_deg_kernel = _make_deg()


def _agg_body(xs_hbm, srcs_hbm, dsts_hbm, zeros_hbm, out_hbm,
              srcv, dstv, rows0, rows1, rows2, rows3, acc,
              sem0, sem1, sem2, sem3):
    c = lax.axis_index("c")
    s = lax.axis_index("s")
    pltpu.sync_copy(srcs_hbm.at[s], srcv)
    pltpu.sync_copy(dsts_hbm.at[s], dstv)

    if True:
        tbl = xs_hbm.at[c]
        pltpu.sync_copy(zeros_hbm, acc.at[pl.ds(s * ROWS_PT, ROWS_PT)])
        plsc.subcore_barrier()

        buf = ((rows0, sem0), (rows1, sem1), (rows2, sem2), (rows3, sem3))
        for b in range(4):
            pltpu.async_copy(tbl.at[srcv.at[b]], buf[b][0], buf[b][1])

        def body(j, carry):
            i = 4 * j
            for b in range(4):
                rb, sb = buf[b]
                pltpu.make_async_copy(tbl.at[srcv.at[i + b]], rb, sb).wait()
                pltpu.sync_copy(rb, acc.at[dstv.at[i + b]], add=True)
                pltpu.async_copy(
                    tbl.at[srcv.at[jnp.minimum(i + b + 4, NCH - 1)]], rb, sb)
            return carry

        lax.fori_loop(0, NCH // 4, body, 0, unroll=False)
        for b in range(4):
            pltpu.make_async_copy(tbl.at[srcv.at[NCH - 1]], buf[b][0],
                                  buf[b][1]).wait()
        plsc.subcore_barrier()
        pltpu.sync_copy(acc.at[pl.ds(s * ROWS_PT, ROWS_PT)],
                        out_hbm.at[c, pl.ds(s * ROWS_PT, ROWS_PT)])


def _make_agg(interpret=False):
    return pl.kernel(
        _agg_body,
        out_type=jax.ShapeDtypeStruct((NC, N_PAD, DH), jnp.float32),
        mesh=_sc_mesh,
        scratch_types=[
            pltpu.VMEM((NCH, CH), jnp.int32),
            pltpu.VMEM((NCH, CH), jnp.int32),
            pltpu.VMEM((CH, DH), jnp.float32),
            pltpu.VMEM((CH, DH), jnp.float32),
            pltpu.VMEM((CH, DH), jnp.float32),
            pltpu.VMEM((CH, DH), jnp.float32),
            pltpu.VMEM_SHARED((N_PAD, DH), jnp.float32),
            pltpu.SemaphoreType.DMA,
            pltpu.SemaphoreType.DMA,
            pltpu.SemaphoreType.DMA,
            pltpu.SemaphoreType.DMA,
        ],
        compiler_params=pltpu.CompilerParams(use_tc_tiling_on_sc=False,
                                             internal_scratch_in_bytes=131072),
        interpret=interpret,
    )


_agg_kernel = _make_agg()


_BLK = 1000


def _tc_scale_body(deg_ref, x_ref, xs_ref, dinv_ref):
    deg = deg_ref[0] + deg_ref[1]
    d0 = deg[:, 0:1]
    dinv = jnp.where(d0 > 0, lax.rsqrt(jnp.maximum(d0, 1e-12)), 0.0)
    dinvb = jnp.broadcast_to(dinv, (_BLK, D))
    dinv_ref[...] = dinvb
    xq = x_ref[...] * dinvb
    for h in range(2):
        xs_ref[h] = xq[:, h * DH:(h + 1) * DH]


def _tc_scale(deg_p, x):
    return pl.pallas_call(
        _tc_scale_body,
        grid=(N_NODES // _BLK,),
        in_specs=[
            pl.BlockSpec((NC, _BLK, 16), lambda i: (0, i, 0)),
            pl.BlockSpec((_BLK, D), lambda i: (i, 0)),
        ],
        out_specs=[
            pl.BlockSpec((NC, _BLK, DH), lambda i: (0, i, 0)),
            pl.BlockSpec((_BLK, D), lambda i: (i, 0)),
        ],
        out_shape=[
            jax.ShapeDtypeStruct((NC, N_NODES, DH), jnp.float32),
            jax.ShapeDtypeStruct((N_NODES, D), jnp.float32),
        ],
    )(deg_p, x)


def _tc_mm_body(last, t_ref, dinv_ref, w_ref, b_ref, out_ref):
    dv = dinv_ref[...]
    z = jnp.concatenate([t_ref[0], t_ref[1]], axis=1) * dv
    h = jnp.dot(z, w_ref[...], preferred_element_type=jnp.float32) + b_ref[...]
    if last:
        out_ref[...] = h
    else:
        h = jnp.maximum(h, 0.0) * dv
        for q in range(2):
            out_ref[q] = h[:, q * DH:(q + 1) * DH]


def _tc_mm(t_p, dinv_b, w, b, last):
    if last:
        out_spec = pl.BlockSpec((_BLK, D), lambda i: (i, 0))
        out_shape = jax.ShapeDtypeStruct((N_NODES, D), jnp.float32)
    else:
        out_spec = pl.BlockSpec((NC, _BLK, DH), lambda i: (0, i, 0))
        out_shape = jax.ShapeDtypeStruct((NC, N_NODES, DH), jnp.float32)
    return pl.pallas_call(
        functools.partial(_tc_mm_body, last),
        grid=(N_NODES // _BLK,),
        in_specs=[
            pl.BlockSpec((NC, _BLK, DH), lambda i: (0, i, 0)),
            pl.BlockSpec((_BLK, D), lambda i: (i, 0)),
            pl.BlockSpec((D, D), lambda i: (0, 0)),
            pl.BlockSpec((1, D), lambda i: (0, 0)),
        ],
        out_specs=out_spec,
        out_shape=out_shape,
    )(t_p, dinv_b, w, b)


def _edge_blocks(edge_index):
    loop = jnp.arange(N_NODES, dtype=jnp.int32)
    src = jnp.concatenate([edge_index[0].astype(jnp.int32), loop])
    dst = jnp.concatenate([edge_index[1].astype(jnp.int32), loop])
    npad = E_PAD - E_TOT
    src_p = jnp.concatenate([src, jnp.zeros((npad,), jnp.int32)])
    dst_p = jnp.concatenate([dst, jnp.full((npad,), N_NODES, jnp.int32)])
    return src_p.reshape(NS, NCH, CH), dst_p.reshape(NS, NCH, CH)


def kernel(x, edge_index, W1, b1, W2, b2):
    srcs, dsts = _edge_blocks(edge_index)

    zeros_q = jnp.zeros((ROWS_PT, DH), jnp.float32)
    zeros_deg = jnp.zeros((ROWS_PT, 16), jnp.float32)
    ones_deg = jnp.ones((CH, 16), jnp.float32)

    deg_p = _deg_kernel(dsts, zeros_deg, ones_deg)
    xs, dinv_b = _tc_scale(deg_p, x)

    t1 = _agg_kernel(xs, srcs, dsts, zeros_q)
    hs = _tc_mm(t1, dinv_b, W1, b1.reshape(1, D), False)

    t2 = _agg_kernel(hs, srcs, dsts, zeros_q)
    out = _tc_mm(t2, dinv_b, W2, b2.reshape(1, D), True)
    return out


# speedup vs baseline: 1.3786x; 1.3786x over previous
import functools

import jax
import jax.numpy as jnp
from jax import lax
from jax.experimental import pallas as pl
from jax.experimental.pallas import tpu as pltpu
from jax.experimental.pallas import tpu_sc as plsc

N_NODES = 10000
D = 128
DQ = D // 4
DH = D // 2
N_EDGES = 320000

NC = 2
NS = 16
CH = 128

E_TOT = N_EDGES + N_NODES
NCH = -(-E_TOT // (NS * CH))
NCH = NCH + (NCH % 2)
E_PAD = NS * NCH * CH
N_PAD = -(-N_NODES // (NS * 8)) * NS * 8
ROWS_PT = N_PAD // NS

_sc_mesh = plsc.VectorSubcoreMesh(core_axis_name="c", subcore_axis_name="s")


def _deg_body(dsts_hbm, zeros_hbm, ones_hbm, deg_hbm, dstv, onesv, acc):
    c = lax.axis_index("c")
    s = lax.axis_index("s")
    pltpu.sync_copy(zeros_hbm, acc.at[pl.ds(s * ROWS_PT, ROWS_PT)])
    pltpu.sync_copy(dsts_hbm.at[s], dstv)
    pltpu.sync_copy(ones_hbm, onesv)
    plsc.subcore_barrier()

    base = c * (NCH // 2)

    def body(i, carry):
        pltpu.sync_copy(onesv, acc.at[dstv.at[base + i]], add=True)
        return carry

    lax.fori_loop(0, NCH // 2, body, 0, unroll=False)
    plsc.subcore_barrier()
    pltpu.sync_copy(acc.at[pl.ds(s * ROWS_PT, ROWS_PT)],
                    deg_hbm.at[c, pl.ds(s * ROWS_PT, ROWS_PT)])


def _make_deg(interpret=False):
    return pl.kernel(
        _deg_body,
        out_type=jax.ShapeDtypeStruct((NC, N_PAD, 16), jnp.float32),
        mesh=_sc_mesh,
        scratch_types=[
            pltpu.VMEM((NCH, CH), jnp.int32),
            pltpu.VMEM((CH, 16), jnp.float32),
            pltpu.VMEM_SHARED((N_PAD, 16), jnp.float32),
        ],
        compiler_params=pltpu.CompilerParams(use_tc_tiling_on_sc=False),
        interpret=interpret,
    )


_deg_kernel = _make_deg()


def _agg_body(xs_hbm, srcs_hbm, dsts_hbm, zeros_hbm, out_hbm,
              srcv, dstv, rows0, rows1, acc, sem0, sem1):
    c = lax.axis_index("c")
    s = lax.axis_index("s")
    pltpu.sync_copy(srcs_hbm.at[s], srcv)
    pltpu.sync_copy(dsts_hbm.at[s], dstv)

    if True:
        tbl = xs_hbm.at[c]
        pltpu.sync_copy(zeros_hbm, acc.at[pl.ds(s * ROWS_PT, ROWS_PT)])
        plsc.subcore_barrier()

        pltpu.async_copy(tbl.at[srcv.at[0]], rows0, sem0)
        pltpu.async_copy(tbl.at[srcv.at[1]], rows1, sem1)

        def body(j, carry):
            i0 = 2 * j
            i1 = i0 + 1
            pltpu.make_async_copy(tbl.at[srcv.at[i0]], rows0, sem0).wait()
            pltpu.sync_copy(rows0, acc.at[dstv.at[i0]], add=True)
            pltpu.async_copy(tbl.at[srcv.at[jnp.minimum(i0 + 2, NCH - 1)]],
                             rows0, sem0)
            pltpu.make_async_copy(tbl.at[srcv.at[i1]], rows1, sem1).wait()
            pltpu.sync_copy(rows1, acc.at[dstv.at[i1]], add=True)
            pltpu.async_copy(tbl.at[srcv.at[jnp.minimum(i1 + 2, NCH - 1)]],
                             rows1, sem1)
            return carry

        lax.fori_loop(0, NCH // 2, body, 0, unroll=False)
        pltpu.make_async_copy(tbl.at[srcv.at[NCH - 1]], rows0, sem0).wait()
        pltpu.make_async_copy(tbl.at[srcv.at[NCH - 1]], rows1, sem1).wait()
        plsc.subcore_barrier()
        pltpu.sync_copy(acc.at[pl.ds(s * ROWS_PT, ROWS_PT)],
                        out_hbm.at[c, pl.ds(s * ROWS_PT, ROWS_PT)])


def _make_agg(interpret=False):
    return pl.kernel(
        _agg_body,
        out_type=jax.ShapeDtypeStruct((NC, N_PAD, DH), jnp.float32),
        mesh=_sc_mesh,
        scratch_types=[
            pltpu.VMEM((NCH, CH), jnp.int32),
            pltpu.VMEM((NCH, CH), jnp.int32),
            pltpu.VMEM((CH, DH), jnp.float32),
            pltpu.VMEM((CH, DH), jnp.float32),
            pltpu.VMEM_SHARED((N_PAD, DH), jnp.float32),
            pltpu.SemaphoreType.DMA,
            pltpu.SemaphoreType.DMA,
        ],
        compiler_params=pltpu.CompilerParams(use_tc_tiling_on_sc=False,
                                             internal_scratch_in_bytes=131072),
        interpret=interpret,
    )


_agg_kernel = _make_agg()


_BLK = 1000


def _dinv_block(deg_ref):
    deg = deg_ref[0] + deg_ref[1]
    d0 = deg[:, 0:1]
    dinv = jnp.where(d0 > 0, lax.rsqrt(jnp.maximum(d0, 1e-12)), 0.0)
    return jnp.broadcast_to(dinv, (_BLK, D))


def _tc_scale_body(deg_ref, x_ref, xs_ref):
    xq = x_ref[...] * _dinv_block(deg_ref)
    for h in range(2):
        xs_ref[h] = xq[:, h * DH:(h + 1) * DH]


def _tc_scale(deg_p, x):
    return pl.pallas_call(
        _tc_scale_body,
        grid=(N_NODES // _BLK,),
        in_specs=[
            pl.BlockSpec((NC, _BLK, 16), lambda i: (0, i, 0)),
            pl.BlockSpec((_BLK, D), lambda i: (i, 0)),
        ],
        out_specs=pl.BlockSpec((NC, _BLK, DH), lambda i: (0, i, 0)),
        out_shape=jax.ShapeDtypeStruct((NC, N_NODES, DH), jnp.float32),
    )(deg_p, x)


def _tc_mm_body(last, t_ref, deg_ref, w_ref, b_ref, out_ref):
    dv = _dinv_block(deg_ref)
    z = jnp.concatenate([t_ref[0], t_ref[1]], axis=1) * dv
    h = jnp.dot(z, w_ref[...], preferred_element_type=jnp.float32) + b_ref[...]
    if last:
        out_ref[...] = h
    else:
        h = jnp.maximum(h, 0.0) * dv
        for q in range(2):
            out_ref[q] = h[:, q * DH:(q + 1) * DH]


def _tc_mm(t_p, deg_p, w, b, last):
    if last:
        out_spec = pl.BlockSpec((_BLK, D), lambda i: (i, 0))
        out_shape = jax.ShapeDtypeStruct((N_NODES, D), jnp.float32)
    else:
        out_spec = pl.BlockSpec((NC, _BLK, DH), lambda i: (0, i, 0))
        out_shape = jax.ShapeDtypeStruct((NC, N_NODES, DH), jnp.float32)
    return pl.pallas_call(
        functools.partial(_tc_mm_body, last),
        grid=(N_NODES // _BLK,),
        in_specs=[
            pl.BlockSpec((NC, _BLK, DH), lambda i: (0, i, 0)),
            pl.BlockSpec((NC, _BLK, 16), lambda i: (0, i, 0)),
            pl.BlockSpec((D, D), lambda i: (0, 0)),
            pl.BlockSpec((1, D), lambda i: (0, 0)),
        ],
        out_specs=out_spec,
        out_shape=out_shape,
    )(t_p, deg_p, w, b)


def _edge_blocks(edge_index):
    loop = jnp.arange(N_NODES, dtype=jnp.int32)
    src = jnp.concatenate([edge_index[0].astype(jnp.int32), loop])
    dst = jnp.concatenate([edge_index[1].astype(jnp.int32), loop])
    npad = E_PAD - E_TOT
    src_p = jnp.concatenate([src, jnp.zeros((npad,), jnp.int32)])
    dst_p = jnp.concatenate([dst, jnp.full((npad,), N_NODES, jnp.int32)])
    return src_p.reshape(NS, NCH, CH), dst_p.reshape(NS, NCH, CH)


def kernel(x, edge_index, W1, b1, W2, b2):
    srcs, dsts = _edge_blocks(edge_index)

    zeros_q = jnp.zeros((ROWS_PT, DH), jnp.float32)
    zeros_deg = jnp.zeros((ROWS_PT, 16), jnp.float32)
    ones_deg = jnp.ones((CH, 16), jnp.float32)

    deg_p = _deg_kernel(dsts, zeros_deg, ones_deg)
    xs = _tc_scale(deg_p, x)

    t1 = _agg_kernel(xs, srcs, dsts, zeros_q)
    hs = _tc_mm(t1, deg_p, W1, b1.reshape(1, D), False)

    t2 = _agg_kernel(hs, srcs, dsts, zeros_q)
    out = _tc_mm(t2, deg_p, W2, b2.reshape(1, D), True)
    return out


# speedup vs baseline: 1.4084x; 1.0216x over previous
import functools

import jax
import jax.numpy as jnp
from jax import lax
from jax.experimental import pallas as pl
from jax.experimental.pallas import tpu as pltpu
from jax.experimental.pallas import tpu_sc as plsc

N_NODES = 10000
D = 128
DQ = D // 4
DH = D // 2
N_EDGES = 320000

NC = 2
NS = 16
CH = 128

E_TOT = N_EDGES + N_NODES
NCH = -(-E_TOT // (NS * CH))
NCH = NCH + (NCH % 2)
E_PAD = NS * NCH * CH
N_PAD = -(-N_NODES // (NS * 8)) * NS * 8
ROWS_PT = N_PAD // NS

_sc_mesh = plsc.VectorSubcoreMesh(core_axis_name="c", subcore_axis_name="s")


def _deg_body(dsts_hbm, zeros_hbm, ones_hbm, deg_hbm, dstv, onesv, acc):
    c = lax.axis_index("c")
    s = lax.axis_index("s")
    pltpu.sync_copy(zeros_hbm, acc.at[pl.ds(s * ROWS_PT, ROWS_PT)])
    pltpu.sync_copy(dsts_hbm.at[s], dstv)
    pltpu.sync_copy(ones_hbm, onesv)
    plsc.subcore_barrier()

    base = c * (NCH // 2)

    def body(i, carry):
        pltpu.sync_copy(onesv, acc.at[dstv.at[base + i]], add=True)
        return carry

    lax.fori_loop(0, NCH // 2, body, 0, unroll=False)
    plsc.subcore_barrier()
    pltpu.sync_copy(acc.at[pl.ds(s * ROWS_PT, ROWS_PT)],
                    deg_hbm.at[c, pl.ds(s * ROWS_PT, ROWS_PT)])


def _make_deg(interpret=False):
    return pl.kernel(
        _deg_body,
        out_type=jax.ShapeDtypeStruct((NC, N_PAD, 16), jnp.float32),
        mesh=_sc_mesh,
        scratch_types=[
            pltpu.VMEM((NCH, CH), jnp.int32),
            pltpu.VMEM((CH, 16), jnp.float32),
            pltpu.VMEM_SHARED((N_PAD, 16), jnp.float32),
        ],
        compiler_params=pltpu.CompilerParams(use_tc_tiling_on_sc=False),
        interpret=interpret,
    )


_deg_kernel = _make_deg()


def _agg_body(xs_hbm, srcs_hbm, dsts_hbm, zeros_hbm, out_hbm,
              srcv, dstv, rows0, rows1, acc, sem0, sem1):
    c = lax.axis_index("c")
    s = lax.axis_index("s")
    pltpu.sync_copy(srcs_hbm.at[s], srcv)
    pltpu.sync_copy(dsts_hbm.at[s], dstv)

    if True:
        tbl = xs_hbm.at[c]
        pltpu.sync_copy(zeros_hbm, acc.at[pl.ds(s * ROWS_PT, ROWS_PT)])
        plsc.subcore_barrier()

        pltpu.async_copy(tbl.at[srcv.at[0]], rows0, sem0)
        pltpu.async_copy(tbl.at[srcv.at[1]], rows1, sem1)

        def body(j, carry):
            i0 = 2 * j
            i1 = i0 + 1
            pltpu.make_async_copy(tbl.at[srcv.at[i0]], rows0, sem0).wait()
            pltpu.sync_copy(rows0, acc.at[dstv.at[i0]], add=True)
            pltpu.async_copy(tbl.at[srcv.at[jnp.minimum(i0 + 2, NCH - 1)]],
                             rows0, sem0)
            pltpu.make_async_copy(tbl.at[srcv.at[i1]], rows1, sem1).wait()
            pltpu.sync_copy(rows1, acc.at[dstv.at[i1]], add=True)
            pltpu.async_copy(tbl.at[srcv.at[jnp.minimum(i1 + 2, NCH - 1)]],
                             rows1, sem1)
            return carry

        lax.fori_loop(0, NCH // 2, body, 0, unroll=False)
        pltpu.make_async_copy(tbl.at[srcv.at[NCH - 1]], rows0, sem0).wait()
        pltpu.make_async_copy(tbl.at[srcv.at[NCH - 1]], rows1, sem1).wait()
        plsc.subcore_barrier()
        pltpu.sync_copy(acc.at[pl.ds(s * ROWS_PT, ROWS_PT)],
                        out_hbm.at[c, pl.ds(s * ROWS_PT, ROWS_PT)])


def _make_agg(interpret=False):
    return pl.kernel(
        _agg_body,
        out_type=jax.ShapeDtypeStruct((NC, N_PAD, DH), jnp.float32),
        mesh=_sc_mesh,
        scratch_types=[
            pltpu.VMEM((NCH, CH), jnp.int32),
            pltpu.VMEM((NCH, CH), jnp.int32),
            pltpu.VMEM((CH, DH), jnp.float32),
            pltpu.VMEM((CH, DH), jnp.float32),
            pltpu.VMEM_SHARED((N_PAD, DH), jnp.float32),
            pltpu.SemaphoreType.DMA,
            pltpu.SemaphoreType.DMA,
        ],
        compiler_params=pltpu.CompilerParams(use_tc_tiling_on_sc=False,
                                             internal_scratch_in_bytes=131072),
        interpret=interpret,
    )


_agg_kernel = _make_agg()


_BLK = 1000


def _tc_scale_body(deg_ref, x_ref, xs_ref, dinv_ref):
    deg = deg_ref[0] + deg_ref[1]
    d0 = deg[:, 0:1]
    dinv = jnp.where(d0 > 0, lax.rsqrt(jnp.maximum(d0, 1e-12)), 0.0)
    dinvb = jnp.broadcast_to(dinv, (_BLK, D))
    dinv_ref[...] = dinvb
    xq = x_ref[...] * dinvb
    for h in range(2):
        xs_ref[h] = xq[:, h * DH:(h + 1) * DH]


def _tc_scale(deg_p, x):
    return pl.pallas_call(
        _tc_scale_body,
        grid=(N_NODES // _BLK,),
        in_specs=[
            pl.BlockSpec((NC, _BLK, 16), lambda i: (0, i, 0)),
            pl.BlockSpec((_BLK, D), lambda i: (i, 0)),
        ],
        out_specs=[
            pl.BlockSpec((NC, _BLK, DH), lambda i: (0, i, 0)),
            pl.BlockSpec((_BLK, D), lambda i: (i, 0)),
        ],
        out_shape=[
            jax.ShapeDtypeStruct((NC, N_NODES, DH), jnp.float32),
            jax.ShapeDtypeStruct((N_NODES, D), jnp.float32),
        ],
    )(deg_p, x)


def _tc_mm_body(last, t_ref, dinv_ref, w_ref, b_ref, out_ref):
    dv = dinv_ref[...]
    z = jnp.concatenate([t_ref[0], t_ref[1]], axis=1) * dv
    h = jnp.dot(z, w_ref[...], preferred_element_type=jnp.float32) + b_ref[...]
    if last:
        out_ref[...] = h
    else:
        h = jnp.maximum(h, 0.0) * dv
        for q in range(2):
            out_ref[q] = h[:, q * DH:(q + 1) * DH]


def _tc_mm(t_p, dinv_b, w, b, last):
    if last:
        out_spec = pl.BlockSpec((_BLK, D), lambda i: (i, 0))
        out_shape = jax.ShapeDtypeStruct((N_NODES, D), jnp.float32)
    else:
        out_spec = pl.BlockSpec((NC, _BLK, DH), lambda i: (0, i, 0))
        out_shape = jax.ShapeDtypeStruct((NC, N_NODES, DH), jnp.float32)
    return pl.pallas_call(
        functools.partial(_tc_mm_body, last),
        grid=(N_NODES // _BLK,),
        in_specs=[
            pl.BlockSpec((NC, _BLK, DH), lambda i: (0, i, 0)),
            pl.BlockSpec((_BLK, D), lambda i: (i, 0)),
            pl.BlockSpec((D, D), lambda i: (0, 0)),
            pl.BlockSpec((1, D), lambda i: (0, 0)),
        ],
        out_specs=out_spec,
        out_shape=out_shape,
    )(t_p, dinv_b, w, b)


def _edge_blocks(edge_index):
    loop = jnp.arange(N_NODES, dtype=jnp.int32)
    src = jnp.concatenate([edge_index[0].astype(jnp.int32), loop])
    dst = jnp.concatenate([edge_index[1].astype(jnp.int32), loop])
    npad = E_PAD - E_TOT
    src_p = jnp.concatenate([src, jnp.zeros((npad,), jnp.int32)])
    dst_p = jnp.concatenate([dst, jnp.full((npad,), N_NODES, jnp.int32)])
    return src_p.reshape(NS, NCH, CH), dst_p.reshape(NS, NCH, CH)


def kernel(x, edge_index, W1, b1, W2, b2):
    srcs, dsts = _edge_blocks(edge_index)

    zeros_q = jnp.zeros((ROWS_PT, DH), jnp.float32)
    zeros_deg = jnp.zeros((ROWS_PT, 16), jnp.float32)
    ones_deg = jnp.ones((CH, 16), jnp.float32)

    deg_p = _deg_kernel(dsts, zeros_deg, ones_deg)
    xs, dinv_b = _tc_scale(deg_p, x)

    t1 = _agg_kernel(xs, srcs, dsts, zeros_q)
    hs = _tc_mm(t1, dinv_b, W1, b1.reshape(1, D), False)

    t2 = _agg_kernel(hs, srcs, dsts, zeros_q)
    out = _tc_mm(t2, dinv_b, W2, b2.reshape(1, D), True)
    return out
